# Initial kernel scaffold; baseline (speedup 1.0000x reference)
import jax
import jax.numpy as jnp
from jax.experimental import pallas as pl


def kernel(x, params):
    raise NotImplementedError("write your pallas kernel here")


# speedup vs baseline: 2.8086x; 2.8086x over previous
import functools
import math

import jax
import jax.numpy as jnp
from jax.experimental import pallas as pl

D_MODEL = 768
N_HEADS = 12
D_KV = 64
D_FF = 3072
N_EXPERTS = 8
CAP = 394
CAP_PAD = 400
SEQ = 197
SEQ_PAD = 208
BATCH = 8
T_PAD = BATCH * SEQ_PAD
NEG = -1e30


def _dot(a, b):
    return jnp.dot(a, b, preferred_element_type=jnp.float32)


def _dot_t(a, b):
    return jax.lax.dot_general(
        a, b, (((0,), (0,)), ((), ())), preferred_element_type=jnp.float32)


def _ln(z, g, b, eps=1e-6):
    mu = jnp.mean(z, axis=-1, keepdims=True)
    var = jnp.mean((z - mu) ** 2, axis=-1, keepdims=True)
    return (z - mu) / jnp.sqrt(var + eps) * g + b


def _erf(u):
    return jax.lax.erf(u)


def _gelu(u):
    return 0.5 * u * (1.0 + _erf(u * (1.0 / math.sqrt(2.0))))


def _embed_body(x_ref, w_ref, b_ref, o_ref):
    o_ref[:] = _dot(x_ref[:], w_ref[:]) + b_ref[:]


def _embed(patches, w, b):
    n = patches.shape[0]
    return pl.pallas_call(
        _embed_body,
        out_shape=jax.ShapeDtypeStruct((n, D_MODEL), jnp.float32),
    )(patches, w, b.reshape(1, D_MODEL))


def _attn_body(h_ref, g_ref, b_ref, wqkv_ref, bqkv_ref, wo_ref, bo_ref, o_ref):
    h = h_ref[:]
    z = _ln(h, g_ref[:], b_ref[:])
    qkv = _dot(z, wqkv_ref[:]) + bqkv_ref[:]
    scale = 1.0 / math.sqrt(D_KV)
    cols = jax.lax.broadcasted_iota(jnp.int32, (SEQ_PAD, SEQ_PAD), 1)
    mask = jnp.where(cols < SEQ, 0.0, NEG)
    outs = []
    for hd in range(N_HEADS):
        q = qkv[:, hd * D_KV:(hd + 1) * D_KV]
        k = qkv[:, 768 + hd * D_KV:768 + (hd + 1) * D_KV]
        v = qkv[:, 1536 + hd * D_KV:1536 + (hd + 1) * D_KV]
        s = jax.lax.dot_general(
            q, k, (((1,), (1,)), ((), ())),
            preferred_element_type=jnp.float32) * scale + mask
        m = jnp.max(s, axis=-1, keepdims=True)
        e = jnp.exp(s - m)
        att = e / jnp.sum(e, axis=-1, keepdims=True)
        outs.append(_dot(att, v))
    o = jnp.concatenate(outs, axis=-1)
    o_ref[:] = h + _dot(o, wo_ref[:]) + bo_ref[:]


def _attn_block(h, blk):
    spec_h = pl.BlockSpec((SEQ_PAD, D_MODEL), lambda i: (i, 0))
    full = lambda *s: pl.BlockSpec(s, lambda i: tuple(0 for _ in s))
    return pl.pallas_call(
        _attn_body,
        grid=(BATCH,),
        in_specs=[
            spec_h,
            full(1, D_MODEL), full(1, D_MODEL),
            full(D_MODEL, 3 * D_MODEL), full(1, 3 * D_MODEL),
            full(D_MODEL, D_MODEL), full(1, D_MODEL),
        ],
        out_specs=spec_h,
        out_shape=jax.ShapeDtypeStruct((T_PAD, D_MODEL), jnp.float32),
    )(h, blk['ln1_g'].reshape(1, -1), blk['ln1_b'].reshape(1, -1),
      blk['W_qkv'], blk['b_qkv'].reshape(1, -1),
      blk['W_o'], blk['b_o'].reshape(1, -1))


FF_SPLIT = 2
FF_BLK = D_FF // FF_SPLIT


def _ffn_body(h_ref, g_ref, b_ref, w1_ref, b1_ref, w2_ref, b2_ref, o_ref):
    f = pl.program_id(0)
    h = h_ref[:]
    z = _ln(h, g_ref[:], b_ref[:])
    h1 = _gelu(_dot(z, w1_ref[:]) + b1_ref[:])
    contrib = _dot(h1, w2_ref[:])

    @pl.when(f == 0)
    def _():
        o_ref[:] = h + contrib + b2_ref[:]

    @pl.when(f > 0)
    def _():
        o_ref[:] = o_ref[:] + contrib


def _ffn_block(h, blk):
    p = blk['ffn']
    full = lambda *s: pl.BlockSpec(s, lambda i: tuple(0 for _ in s))
    return pl.pallas_call(
        _ffn_body,
        grid=(FF_SPLIT,),
        in_specs=[
            full(T_PAD, D_MODEL),
            full(1, D_MODEL), full(1, D_MODEL),
            pl.BlockSpec((D_MODEL, FF_BLK), lambda f: (0, f)),
            pl.BlockSpec((1, FF_BLK), lambda f: (0, f)),
            pl.BlockSpec((FF_BLK, D_MODEL), lambda f: (f, 0)),
            full(1, D_MODEL),
        ],
        out_specs=full(T_PAD, D_MODEL),
        out_shape=jax.ShapeDtypeStruct((T_PAD, D_MODEL), jnp.float32),
    )(h, blk['ln2_g'].reshape(1, -1), blk['ln2_b'].reshape(1, -1),
      p['W1'], p['b1'].reshape(1, -1), p['W2'], p['b2'].reshape(1, -1))


def _route_body(h_ref, g_ref, b_ref, wg_ref, z_ref, v_ref):
    h = h_ref[:]
    z = _ln(h, g_ref[:], b_ref[:])
    z_ref[:] = z
    logits = _dot(z, wg_ref[:])
    m = jnp.max(logits, axis=-1, keepdims=True)
    e = jnp.exp(logits - m)
    gates = e / jnp.sum(e, axis=-1, keepdims=True)
    iota_e = jax.lax.broadcasted_iota(
        jnp.int32, (T_PAD, N_EXPERTS), 1).astype(jnp.float32)
    g0 = jnp.max(gates, axis=-1, keepdims=True)
    e0 = jnp.min(jnp.where(gates == g0, iota_e, 1e9), axis=-1, keepdims=True)
    gates1 = jnp.where(iota_e == e0, -1.0, gates)
    g1 = jnp.max(gates1, axis=-1, keepdims=True)
    e1 = jnp.min(jnp.where(gates1 == g1, iota_e, 1e9), axis=-1, keepdims=True)

    t_idx = jax.lax.broadcasted_iota(jnp.int32, (T_PAD, 1), 0)
    real = (t_idx % SEQ_PAD < SEQ).astype(jnp.float32)

    mask0 = (iota_e == e0).astype(jnp.float32) * real
    mask1 = (iota_e == e1).astype(jnp.float32) * real
    rows = jax.lax.broadcasted_iota(jnp.int32, (T_PAD, T_PAD), 0)
    colsq = jax.lax.broadcasted_iota(jnp.int32, (T_PAD, T_PAD), 1)
    tri = (colsq < rows).astype(jnp.float32)
    cum0 = _dot(tri, mask0)
    cum1 = _dot(tri, mask1) + jnp.sum(mask0, axis=0, keepdims=True)
    pos0 = jnp.sum(cum0 * mask0, axis=-1, keepdims=True)
    pos1 = jnp.sum(cum1 * mask1, axis=-1, keepdims=True)
    keep0 = (pos0 < CAP) * real
    keep1 = (pos1 < CAP) * real
    p0 = jnp.where(keep0 > 0.0, pos0, CAP_PAD + 1.0)
    p1 = jnp.where(keep1 > 0.0, pos1, CAP_PAD + 1.0)
    denom = g0 + g1 + 1e-9
    w0 = g0 / denom * keep0
    w1 = g1 / denom * keep1
    v_ref[:] = jnp.concatenate([e0, p0, w0, e1, p1, w1,
                                jnp.zeros((T_PAD, 2), jnp.float32)], axis=-1)


def _route(h, blk):
    p = blk['moe']
    return pl.pallas_call(
        _route_body,
        out_shape=[
            jax.ShapeDtypeStruct((T_PAD, D_MODEL), jnp.float32),
            jax.ShapeDtypeStruct((T_PAD, 8), jnp.float32),
        ],
    )(h, blk['ln2_g'].reshape(1, -1), blk['ln2_b'].reshape(1, -1), p['Wg'])


def _expert_body(z_ref, h_ref, v_ref, w1_ref, b1_ref, w2_ref, b2_ref, o_ref):
    e = pl.program_id(0)
    f = pl.program_id(1)
    ef = e.astype(jnp.float32)
    v = v_ref[:]
    e0 = v[:, 0:1]
    p0 = v[:, 1:2]
    w0 = v[:, 2:3]
    e1 = v[:, 3:4]
    p1 = v[:, 4:5]
    w1 = v[:, 5:6]
    iota_c = jax.lax.broadcasted_iota(
        jnp.int32, (T_PAD, CAP_PAD), 1).astype(jnp.float32)
    hit0 = (e0 == ef).astype(jnp.float32) * (iota_c == p0).astype(jnp.float32)
    hit1 = (e1 == ef).astype(jnp.float32) * (iota_c == p1).astype(jnp.float32)
    disp = hit0 + hit1
    buf = _dot_t(disp, z_ref[:])
    hmid = _gelu(_dot(buf, w1_ref[0]) + b1_ref[0])
    eo = _dot(hmid, w2_ref[0])
    eo = jnp.where(f == 0, eo + b2_ref[0], eo)
    comb = hit0 * w0 + hit1 * w1
    contrib = _dot(comb, eo)
    first = (e == 0) & (f == 0)

    @pl.when(first)
    def _():
        o_ref[:] = h_ref[:] + contrib

    @pl.when(jnp.logical_not(first))
    def _():
        o_ref[:] = o_ref[:] + contrib


def _moe_block(h, blk):
    p = blk['moe']
    z, v = _route(h, blk)
    full = lambda *s: pl.BlockSpec(s, lambda *i: tuple(0 for _ in s))
    return pl.pallas_call(
        _expert_body,
        grid=(N_EXPERTS, FF_SPLIT),
        in_specs=[
            full(T_PAD, D_MODEL),
            full(T_PAD, D_MODEL),
            full(T_PAD, 8),
            pl.BlockSpec((1, D_MODEL, FF_BLK), lambda e, f: (e, 0, f)),
            pl.BlockSpec((1, 1, FF_BLK), lambda e, f: (e, 0, f)),
            pl.BlockSpec((1, FF_BLK, D_MODEL), lambda e, f: (e, f, 0)),
            pl.BlockSpec((1, 1, D_MODEL), lambda e, f: (e, 0, 0)),
        ],
        out_specs=full(T_PAD, D_MODEL),
        out_shape=jax.ShapeDtypeStruct((T_PAD, D_MODEL), jnp.float32),
    )(z, h, v, p['We1'], p['be1'].reshape(N_EXPERTS, 1, D_FF),
      p['We2'], p['be2'].reshape(N_EXPERTS, 1, D_MODEL))


def _final_body(h_ref, g_ref, b_ref, wh_ref, bh_ref, o_ref):
    z = _ln(h_ref[:], g_ref[:], b_ref[:])
    rows = jax.lax.broadcasted_iota(jnp.int32, (BATCH, T_PAD), 1)
    b_of_t = rows // SEQ_PAD
    s_of_t = rows % SEQ_PAD
    bidx = jax.lax.broadcasted_iota(jnp.int32, (BATCH, T_PAD), 0)
    sel = ((b_of_t == bidx) & (s_of_t < SEQ)).astype(jnp.float32) / SEQ
    pooled = _dot(sel, z)
    o_ref[:] = _dot(pooled, wh_ref[:]) + bh_ref[:]


def _final(h, g, b, wh, bh):
    n_cls = wh.shape[1]
    return pl.pallas_call(
        _final_body,
        out_shape=jax.ShapeDtypeStruct((BATCH, n_cls), jnp.float32),
    )(h, g.reshape(1, -1), b.reshape(1, -1), wh, bh.reshape(1, -1))


def kernel(x, params):
    p = params
    hw = 224 // 16
    patches = x.reshape(BATCH, 3, hw, 16, hw, 16)
    patches = patches.transpose(0, 2, 4, 1, 3, 5).reshape(BATCH * hw * hw, 768)
    emb = _embed(patches, p['patch_W'], p['patch_b'])
    emb = emb.reshape(BATCH, hw * hw, D_MODEL)
    cls = jnp.broadcast_to(p['cls_tok'], (BATCH, 1, D_MODEL))
    h = jnp.concatenate([cls, emb], axis=1) + p['pos_emb']
    h = jnp.pad(h, ((0, 0), (0, SEQ_PAD - SEQ), (0, 0)))
    h = h.reshape(T_PAD, D_MODEL)
    for blk in p['blocks']:
        h = _attn_block(h, blk)
        if 'ffn' in blk:
            h = _ffn_block(h, blk)
        else:
            h = _moe_block(h, blk)
    return _final(h, p['ln_f_g'], p['ln_f_b'], p['head_W'], p['head_b'])
